# baseline (device time: 241370 ns/iter reference)
import jax
import jax.numpy as jnp
from jax import lax
from jax.experimental import pallas as pl
from jax.experimental.pallas import tpu as pltpu

N_DEV = 32
SQ = 1024
SKV = 1024
HQ = 8
DH = 128
D_MODEL = 1024
HD = HQ * DH
CHUNK = SQ // N_DEV
SCALE = 0.08838834764831843


def kernel(x, Wq, K_ext, V_ext, Wo):
    my = lax.axis_index("i")
    wq_s = lax.dynamic_slice(Wq, (0, my * HD), (D_MODEL, HD)).astype(jnp.bfloat16)
    wo_s = lax.dynamic_slice(Wo, (my * HD, 0), (HD, D_MODEL)).astype(jnp.bfloat16)
    xb = x[0].astype(jnp.bfloat16)
    kb = K_ext[0].astype(jnp.bfloat16)
    vb = V_ext[0].astype(jnp.bfloat16)

    def body(x_ref, wq_ref, k_ref, v_ref, wo_ref, out_ref,
             ctx_ref, partial, comm_rs, comm_ag,
             send_rs, recv_rs, send_ag, recv_ag):
        p = lax.axis_index("i")
        right = lax.rem(p + 1, N_DEV)
        left = lax.rem(p + N_DEV - 1, N_DEV)

        q_all = lax.dot_general(
            x_ref[...], wq_ref[...], (((1,), (0,)), ((), ())),
            preferred_element_type=jnp.float32,
        ).astype(jnp.bfloat16)

        qi = lax.broadcasted_iota(jnp.int32, (SQ, SKV), 0)
        ki = lax.broadcasted_iota(jnp.int32, (SQ, SKV), 1)
        mask = (jnp.abs(qi - ki) <= 128) | (ki < 32) | (qi < 32)

        for h in range(HQ):
            q = q_all[:, h * DH:(h + 1) * DH]
            k = k_ref[:, h, :]
            s = lax.dot_general(q, k, (((1,), (1,)), ((), ())),
                                preferred_element_type=jnp.float32) * SCALE
            s = jnp.where(mask, s, jnp.float32(-1e9))
            m = jnp.max(s, axis=1, keepdims=True)
            e = jnp.exp(s - m)
            w = (e / jnp.sum(e, axis=1, keepdims=True)).astype(jnp.bfloat16)
            c = lax.dot_general(w, v_ref[:, h, :], (((1,), (0,)), ((), ())),
                                preferred_element_type=jnp.float32)
            ctx_ref[:, h * DH:(h + 1) * DH] = c.astype(jnp.bfloat16)

        partial[...] = lax.dot_general(
            ctx_ref[...], wo_ref[...], (((1,), (0,)), ((), ())),
            preferred_element_type=jnp.float32)

        bar = pltpu.get_barrier_semaphore()
        pl.semaphore_signal(bar, inc=1, device_id=(left,),
                            device_id_type=pl.DeviceIdType.MESH)
        pl.semaphore_signal(bar, inc=1, device_id=(right,),
                            device_id_type=pl.DeviceIdType.MESH)
        pl.semaphore_wait(bar, 2)

        for s in range(N_DEV - 1):
            c_send = lax.rem(p + N_DEV - s, N_DEV)
            rdma = pltpu.make_async_remote_copy(
                src_ref=partial.at[pl.ds(c_send * CHUNK, CHUNK), :],
                dst_ref=comm_rs.at[s],
                send_sem=send_rs.at[s],
                recv_sem=recv_rs.at[s],
                device_id=(right,),
                device_id_type=pl.DeviceIdType.MESH,
            )
            rdma.start()
            rdma.wait()
            c_recv = lax.rem(p + N_DEV - s - 1, N_DEV)
            rows = pl.ds(c_recv * CHUNK, CHUNK)
            partial[rows, :] = partial[rows, :] + comm_rs[s]

        own = lax.rem(p + 1, N_DEV)
        for s in range(N_DEV - 1):
            src = (partial.at[pl.ds(own * CHUNK, CHUNK), :] if s == 0
                   else comm_ag.at[s - 1])
            rdma = pltpu.make_async_remote_copy(
                src_ref=src,
                dst_ref=comm_ag.at[s],
                send_sem=send_ag.at[s],
                recv_sem=recv_ag.at[s],
                device_id=(right,),
                device_id_type=pl.DeviceIdType.MESH,
            )
            rdma.start()
            rdma.wait()
            c_recv = lax.rem(p + N_DEV - s, N_DEV)
            out_ref[0, pl.ds(c_recv * CHUNK, CHUNK), :] = comm_ag[s]
        out_ref[0, pl.ds(own * CHUNK, CHUNK), :] = (
            partial[pl.ds(own * CHUNK, CHUNK), :])

    return pl.pallas_call(
        body,
        out_shape=jax.ShapeDtypeStruct((1, SQ, D_MODEL), jnp.float32),
        in_specs=[pl.BlockSpec(memory_space=pltpu.VMEM)] * 5,
        out_specs=pl.BlockSpec(memory_space=pltpu.VMEM),
        scratch_shapes=[
            pltpu.VMEM((SQ, HD), jnp.bfloat16),
            pltpu.VMEM((SQ, D_MODEL), jnp.float32),
            pltpu.VMEM((N_DEV - 1, CHUNK, D_MODEL), jnp.float32),
            pltpu.VMEM((N_DEV - 1, CHUNK, D_MODEL), jnp.float32),
            pltpu.SemaphoreType.DMA((N_DEV - 1,)),
            pltpu.SemaphoreType.DMA((N_DEV - 1,)),
            pltpu.SemaphoreType.DMA((N_DEV - 1,)),
            pltpu.SemaphoreType.DMA((N_DEV - 1,)),
        ],
        compiler_params=pltpu.CompilerParams(collective_id=0),
    )(xb, wq_s, kb, vb, wo_s)


# device time: 95423 ns/iter; 2.5295x vs baseline; 2.5295x over previous
import jax
import jax.numpy as jnp
from jax import lax
from jax.experimental import pallas as pl
from jax.experimental.pallas import tpu as pltpu

N_DEV = 32
SQ = 1024
SKV = 1024
HQ = 8
DH = 128
D_MODEL = 1024
HD = HQ * DH
CHUNK = SQ // N_DEV
SCALE = 0.08838834764831843


def kernel(x, Wq, K_ext, V_ext, Wo):
    my = lax.axis_index("i")
    wq_s = lax.dynamic_slice(Wq, (0, my * HD), (D_MODEL, HD)).astype(jnp.bfloat16)
    wo_s = lax.dynamic_slice(Wo, (my * HD, 0), (HD, D_MODEL)).astype(jnp.bfloat16)
    xb = x[0].astype(jnp.bfloat16)
    kb = K_ext[0].astype(jnp.bfloat16)
    vb = V_ext[0].astype(jnp.bfloat16)

    def body(x_ref, wq_ref, k_ref, v_ref, wo_ref, out_ref,
             ctx_ref, partial, pbf, comm1, comm2,
             send1, recv1, send2, recv2):
        p = lax.axis_index("i")

        q_all = lax.dot_general(
            x_ref[...], wq_ref[...], (((1,), (0,)), ((), ())),
            preferred_element_type=jnp.float32,
        ).astype(jnp.bfloat16)

        qi = lax.broadcasted_iota(jnp.int32, (SQ, SKV), 0)
        ki = lax.broadcasted_iota(jnp.int32, (SQ, SKV), 1)
        mask = (jnp.abs(qi - ki) <= 128) | (ki < 32) | (qi < 32)

        for h in range(HQ):
            q = q_all[:, h * DH:(h + 1) * DH]
            k = k_ref[:, h, :]
            s = lax.dot_general(q, k, (((1,), (1,)), ((), ())),
                                preferred_element_type=jnp.float32) * SCALE
            s = jnp.where(mask, s, jnp.float32(-1e9))
            m = jnp.max(s, axis=1, keepdims=True)
            e = jnp.exp(s - m)
            w = (e / jnp.sum(e, axis=1, keepdims=True)).astype(jnp.bfloat16)
            c = lax.dot_general(w, v_ref[:, h, :], (((1,), (0,)), ((), ())),
                                preferred_element_type=jnp.float32)
            ctx_ref[:, h * DH:(h + 1) * DH] = c.astype(jnp.bfloat16)

        partial[...] = lax.dot_general(
            ctx_ref[...], wo_ref[...], (((1,), (0,)), ((), ())),
            preferred_element_type=jnp.float32)
        pbf[...] = partial[...].astype(jnp.bfloat16)

        bar = pltpu.get_barrier_semaphore()
        for j in range(1, N_DEV):
            pl.semaphore_signal(bar, inc=1,
                                device_id=(lax.rem(p + j, N_DEV),),
                                device_id_type=pl.DeviceIdType.MESH)
        pl.semaphore_wait(bar, N_DEV - 1)

        sends1 = []
        for j in range(1, N_DEV):
            d = lax.rem(p + j, N_DEV)
            snd = pltpu.make_async_remote_copy(
                src_ref=pbf.at[pl.ds(d * CHUNK, CHUNK), :],
                dst_ref=comm1.at[p],
                send_sem=send1.at[d],
                recv_sem=recv1.at[p],
                device_id=(d,),
                device_id_type=pl.DeviceIdType.MESH,
            )
            snd.start()
            sends1.append(snd)

        comm1[pl.ds(p, 1)] = jnp.zeros((1, CHUNK, D_MODEL), jnp.bfloat16)

        for j in range(1, N_DEV):
            d = lax.rem(p + j, N_DEV)
            rcv = pltpu.make_async_remote_copy(
                src_ref=pbf.at[pl.ds(0, CHUNK), :],
                dst_ref=comm1.at[d],
                send_sem=send1.at[d],
                recv_sem=recv1.at[d],
                device_id=(d,),
                device_id_type=pl.DeviceIdType.MESH,
            )
            rcv.wait_recv()

        reduced = (partial[pl.ds(p * CHUNK, CHUNK), :]
                   + jnp.sum(comm1[...].astype(jnp.float32), axis=0))
        comm2[pl.ds(p, 1)] = reduced.astype(jnp.bfloat16)[None]

        sends2 = []
        for j in range(1, N_DEV):
            d = lax.rem(p + j, N_DEV)
            snd = pltpu.make_async_remote_copy(
                src_ref=comm2.at[p],
                dst_ref=comm2.at[p],
                send_sem=send2.at[d],
                recv_sem=recv2.at[p],
                device_id=(d,),
                device_id_type=pl.DeviceIdType.MESH,
            )
            snd.start()
            sends2.append(snd)

        for j in range(1, N_DEV):
            d = lax.rem(p + j, N_DEV)
            rcv = pltpu.make_async_remote_copy(
                src_ref=comm2.at[p],
                dst_ref=comm2.at[d],
                send_sem=send2.at[d],
                recv_sem=recv2.at[d],
                device_id=(d,),
                device_id_type=pl.DeviceIdType.MESH,
            )
            rcv.wait_recv()

        out_ref[0, :, :] = jnp.reshape(
            comm2[...], (SQ, D_MODEL)).astype(jnp.float32)

        for snd in sends1:
            snd.wait_send()
        for snd in sends2:
            snd.wait_send()

    return pl.pallas_call(
        body,
        out_shape=jax.ShapeDtypeStruct((1, SQ, D_MODEL), jnp.float32),
        in_specs=[pl.BlockSpec(memory_space=pltpu.VMEM)] * 5,
        out_specs=pl.BlockSpec(memory_space=pltpu.VMEM),
        scratch_shapes=[
            pltpu.VMEM((SQ, HD), jnp.bfloat16),
            pltpu.VMEM((SQ, D_MODEL), jnp.float32),
            pltpu.VMEM((SQ, D_MODEL), jnp.bfloat16),
            pltpu.VMEM((N_DEV, CHUNK, D_MODEL), jnp.bfloat16),
            pltpu.VMEM((N_DEV, CHUNK, D_MODEL), jnp.bfloat16),
            pltpu.SemaphoreType.DMA((N_DEV,)),
            pltpu.SemaphoreType.DMA((N_DEV,)),
            pltpu.SemaphoreType.DMA((N_DEV,)),
            pltpu.SemaphoreType.DMA((N_DEV,)),
        ],
        compiler_params=pltpu.CompilerParams(collective_id=0),
    )(xb, wq_s, kb, vb, wo_s)
